# Initial kernel scaffold; baseline (speedup 1.0000x reference)
import jax
import jax.numpy as jnp
from jax.experimental import pallas as pl


def kernel(features):
    raise NotImplementedError("write your pallas kernel here")


# speedup vs baseline: 3.8887x; 3.8887x over previous
import numpy as np
import jax
import jax.numpy as jnp
from jax import lax
from jax.experimental import pallas as pl

_SCALES = (4, 8, 16)
_K = 4
_B, _H, _W, _D = 64, 32, 32, 192
_NS = tuple(s * s for s in _SCALES)
_OFFS = (0, 16, 80)
_NTOT = sum(_NS)
_BD = _B * _D


def _pool_matrix() -> np.ndarray:
    blocks = []
    for s in _SCALES:
        bs = _H // s
        a = np.kron(np.eye(s, dtype=np.float32), np.ones((1, bs), np.float32))
        blocks.append(np.kron(a, a) / np.float32(bs * bs))
    return np.concatenate(blocks, axis=0)


def _pool_body(x_ref, p_ref, out_ref):
    p = p_ref[...]
    x0 = x_ref[0].reshape(_H * _W, _D)
    x1 = x_ref[1].reshape(_H * _W, _D)
    r0 = jnp.dot(p, x0, preferred_element_type=jnp.float32)
    r1 = jnp.dot(p, x1, preferred_element_type=jnp.float32)
    out_ref[...] = jnp.concatenate([r0, r1], axis=1)


def _sample_body(flat_ref, oh4_ref, g4_ref, oh8_ref, g8_ref, oh16_ref,
                 g16_ref, exp_ref, bo_ref, out_ref):
    exp_m = exp_ref[...]
    bo_m = bo_ref[...]

    def run_scale(flat, oh0, g_ref, slot):
        n = flat.shape[0]
        oh = oh0
        min_d2 = None
        for t in range(_K):
            oh_exp = jnp.dot(oh, exp_m, preferred_element_type=jnp.float32)
            cexp = jnp.sum(oh_exp * flat, axis=0, keepdims=True)
            out_ref[slot + t:slot + t + 1, :] = cexp
            if t == _K - 1:
                break
            diff = flat - cexp
            d2 = jnp.dot(diff * diff, bo_m,
                         preferred_element_type=jnp.float32)
            min_d2 = d2 if min_d2 is None else jnp.minimum(min_d2, d2)
            md = jnp.sqrt(jnp.maximum(min_d2, 0.0))
            p = md * md
            s = jnp.sum(p, axis=0, keepdims=True)
            pn = p / (s + 1e-8)
            sc = jnp.log(pn + 1e-30) + g_ref[t]
            m = jnp.max(sc, axis=0, keepdims=True)
            niota = lax.broadcasted_iota(jnp.int32, (n, _B), 0)
            cand = jnp.where(sc == m, niota, n)
            sel = jnp.min(cand, axis=0, keepdims=True)
            oh = (niota == sel).astype(jnp.float32)
        return

    run_scale(flat_ref[_OFFS[0]:_OFFS[0] + _NS[0], :], oh4_ref[...], g4_ref, 0)
    run_scale(flat_ref[_OFFS[1]:_OFFS[1] + _NS[1], :], oh8_ref[...], g8_ref, 4)
    run_scale(flat_ref[_OFFS[2]:_OFFS[2] + _NS[2], :], oh16_ref[...], g16_ref, 8)


def _rand_setup():
    base_key = jax.random.key(42)
    oh0s, gs = [], []
    for si, s in enumerate(_SCALES):
        n = s * s
        keys = jax.random.split(jax.random.fold_in(base_key, si), _B)

        def draws(kk, n=n):
            key, sub = jax.random.split(kk)
            i0 = jax.random.randint(sub, (), 0, n)
            g = []
            for _ in range(_K - 1):
                key, sub = jax.random.split(key)
                g.append(jax.random.gumbel(sub, (n,)))
            return i0, jnp.stack(g)

        i0s, gstack = jax.vmap(draws)(keys)
        oh0 = (i0s[None, :] == jnp.arange(n)[:, None]).astype(jnp.float32)
        gs.append(jnp.transpose(gstack, (1, 2, 0)))
        oh0s.append(oh0)
    return oh0s, gs


def kernel(features):
    p_mat = jnp.asarray(_pool_matrix())
    expander = jnp.asarray(
        np.kron(np.eye(_B, dtype=np.float32), np.ones((1, _D), np.float32)))
    blockones = expander.T

    flat = pl.pallas_call(
        _pool_body,
        grid=(_B // 2,),
        in_specs=[
            pl.BlockSpec((2, _H, _W, _D), lambda i: (i, 0, 0, 0)),
            pl.BlockSpec((_NTOT, _H * _W), lambda i: (0, 0)),
        ],
        out_specs=pl.BlockSpec((_NTOT, 2 * _D), lambda i: (0, i)),
        out_shape=jax.ShapeDtypeStruct((_NTOT, _BD), jnp.float32),
    )(features, p_mat)

    oh0s, gs = _rand_setup()

    centers = pl.pallas_call(
        _sample_body,
        out_shape=jax.ShapeDtypeStruct((3 * _K, _BD), jnp.float32),
    )(flat, oh0s[0], gs[0], oh0s[1], gs[1], oh0s[2], gs[2],
      expander, blockones)

    return centers.reshape(3 * _K, _B, _D).transpose(1, 0, 2)


# speedup vs baseline: 4.7751x; 1.2279x over previous
import numpy as np
import jax
import jax.numpy as jnp
from jax import lax
from jax.experimental import pallas as pl

_SCALES = (4, 8, 16)
_K = 4
_B, _H, _W, _D = 64, 32, 32, 192
_NS = tuple(s * s for s in _SCALES)
_OFFS = (0, 16, 80)
_NTOT = sum(_NS)
_BD = _B * _D


def _pool_matrix() -> np.ndarray:
    blocks = []
    for s in _SCALES:
        bs = _H // s
        a = np.kron(np.eye(s, dtype=np.float32), np.ones((1, bs), np.float32))
        blocks.append(np.kron(a, a) / np.float32(bs * bs))
    return np.concatenate(blocks, axis=0)


def _pool_body(x_ref, p_ref, out_ref):
    p = p_ref[...]
    x0 = x_ref[0].reshape(_H * _W, _D)
    x1 = x_ref[1].reshape(_H * _W, _D)
    r0 = jnp.dot(p, x0, preferred_element_type=jnp.float32)
    r1 = jnp.dot(p, x1, preferred_element_type=jnp.float32)
    out_ref[...] = jnp.concatenate([r0, r1], axis=1)


def _sample_body(flat_ref, oh4_ref, g4_ref, oh8_ref, g8_ref, oh16_ref,
                 g16_ref, exp_ref, bo_ref, ones1b_ref, eyeb_ref, out_ref):
    exp_m = exp_ref[...]
    bo_m = bo_ref[...]
    ones1b = ones1b_ref[...]
    eyeb = eyeb_ref[...]

    def run_scale(flat, oh0t, g_ref, slot):
        n = flat.shape[0]
        oht = oh0t
        min_d2 = None
        for t in range(_K):
            call = jnp.dot(oht, flat, preferred_element_type=jnp.float32)
            cexp = jnp.dot(ones1b, call * exp_m,
                           preferred_element_type=jnp.float32)
            out_ref[slot + t:slot + t + 1, :] = cexp
            if t == _K - 1:
                break
            diff = flat - cexp
            d2 = jnp.dot(diff * diff, bo_m,
                         preferred_element_type=jnp.float32)
            min_d2 = d2 if min_d2 is None else jnp.minimum(min_d2, d2)
            md = jnp.sqrt(jnp.maximum(min_d2, 0.0))
            p = md * md
            s = jnp.sum(p, axis=0, keepdims=True)
            pn = p / (s + 1e-8)
            sc = jnp.log(pn + 1e-30) + g_ref[t]
            m = jnp.max(sc, axis=0, keepdims=True)
            niota = lax.broadcasted_iota(jnp.int32, (n, _B), 0)
            cand = jnp.where(sc == m, niota, n)
            sel = jnp.min(cand, axis=0, keepdims=True)
            selt = jnp.sum(eyeb * sel.astype(jnp.float32), axis=1,
                           keepdims=True).astype(jnp.int32)
            niota_t = lax.broadcasted_iota(jnp.int32, (_B, n), 1)
            oht = (niota_t == selt).astype(jnp.float32)
        return

    run_scale(flat_ref[_OFFS[0]:_OFFS[0] + _NS[0], :], oh4_ref[...], g4_ref, 0)
    run_scale(flat_ref[_OFFS[1]:_OFFS[1] + _NS[1], :], oh8_ref[...], g8_ref, 4)
    run_scale(flat_ref[_OFFS[2]:_OFFS[2] + _NS[2], :], oh16_ref[...], g16_ref, 8)


def _rand_setup():
    base_key = jax.random.key(42)
    oh0s, gs = [], []
    for si, s in enumerate(_SCALES):
        n = s * s
        keys = jax.random.split(jax.random.fold_in(base_key, si), _B)

        def draws(kk, n=n):
            key, sub = jax.random.split(kk)
            i0 = jax.random.randint(sub, (), 0, n)
            g = []
            for _ in range(_K - 1):
                key, sub = jax.random.split(key)
                g.append(jax.random.gumbel(sub, (n,)))
            return i0, jnp.stack(g)

        i0s, gstack = jax.vmap(draws)(keys)
        oh0t = (i0s[:, None] == jnp.arange(n)[None, :]).astype(jnp.float32)
        gs.append(np.asarray(jnp.transpose(gstack, (1, 2, 0))))
        oh0s.append(np.asarray(oh0t))
    return oh0s, gs


_OH0S, _GS = _rand_setup()


def kernel(features):
    p_mat = jnp.asarray(_pool_matrix())
    expander = jnp.asarray(
        np.kron(np.eye(_B, dtype=np.float32), np.ones((1, _D), np.float32)))
    blockones = expander.T

    flat = pl.pallas_call(
        _pool_body,
        grid=(_B // 2,),
        in_specs=[
            pl.BlockSpec((2, _H, _W, _D), lambda i: (i, 0, 0, 0)),
            pl.BlockSpec((_NTOT, _H * _W), lambda i: (0, 0)),
        ],
        out_specs=pl.BlockSpec((_NTOT, 2 * _D), lambda i: (0, i)),
        out_shape=jax.ShapeDtypeStruct((_NTOT, _BD), jnp.float32),
    )(features, p_mat)

    oh0s = [jnp.asarray(x) for x in _OH0S]
    gs = [jnp.asarray(x) for x in _GS]

    ones1b = jnp.ones((1, _B), jnp.float32)
    eyeb = jnp.asarray(np.eye(_B, dtype=np.float32))
    centers = pl.pallas_call(
        _sample_body,
        out_shape=jax.ShapeDtypeStruct((3 * _K, _BD), jnp.float32),
    )(flat, oh0s[0], gs[0], oh0s[1], gs[1], oh0s[2], gs[2],
      expander, blockones, ones1b, eyeb)

    return centers.reshape(3 * _K, _B, _D).transpose(1, 0, 2)
